# Initial kernel scaffold; baseline (speedup 1.0000x reference)
import jax
import jax.numpy as jnp
from jax.experimental import pallas as pl


def kernel(data, target, org_edge_index, emb_tables, gru_Wih, gru_Whh, gru_bih, gru_bhh, enc_W, enc_b, enc_g, enc_beta, dec_W, dec_b, dec_g, dec_beta, gat_W, att_i, att_j, gat_b, gnn_g, gnn_beta, bno_g, bno_beta, out_W, out_b):
    raise NotImplementedError("write your pallas kernel here")


# speedup vs baseline: 140.3621x; 140.3621x over previous
import jax
import jax.numpy as jnp
from jax.experimental import pallas as pl
from jax.experimental.pallas import tpu as pltpu

B, N, W, DIM, H, NPROC = 32, 256, 16, 64, 32, 4
E_ORG = 16384
D = N * W
CBLK = 512
C = D // CBLK
ECHUNK = 512
NCHUNK = E_ORG // ECHUNK


def _ae_body(x_ref, w_ref, b_ref, g_ref, bt_ref, o_ref, zmid):
    l = pl.program_id(0)
    c = pl.program_id(1)

    def layer(zin):
        h = jax.lax.dot_general(zin, w_ref[0], (((1,), (1,)), ((), ())),
                                preferred_element_type=jnp.float32)
        h = h + b_ref[0, 0]
        m = jnp.mean(h, axis=0, keepdims=True)
        v = jnp.mean((h - m) * (h - m), axis=0, keepdims=True)
        return jax.nn.sigmoid(
            (h - m) / jnp.sqrt(v + 1e-5) * g_ref[0, 0] + bt_ref[0, 0])

    @pl.when(l == 0)
    def _():
        zb = layer(x_ref[...])
        zmid[:, pl.ds(c * CBLK, CBLK)] = zb
        o_ref[...] = zb

    @pl.when(l == 1)
    def _():
        o_ref[...] = layer(zmid[...])


def _ae(x, Ws, bs, gs, bts):
    return pl.pallas_call(
        _ae_body,
        grid=(2, C),
        in_specs=[
            pl.BlockSpec((B, D), lambda l, c: (0, 0)),
            pl.BlockSpec((1, CBLK, D), lambda l, c: (l, c, 0)),
            pl.BlockSpec((1, 1, 1, CBLK), lambda l, c: (l, c, 0, 0)),
            pl.BlockSpec((1, 1, 1, CBLK), lambda l, c: (l, c, 0, 0)),
            pl.BlockSpec((1, 1, 1, CBLK), lambda l, c: (l, c, 0, 0)),
        ],
        out_specs=pl.BlockSpec((B, CBLK), lambda l, c: (0, c)),
        out_shape=jax.ShapeDtypeStruct((B, D), jnp.float32),
        scratch_shapes=[pltpu.VMEM((B, D), jnp.float32)],
    )(x, Ws, bs.reshape(2, C, 1, CBLK), gs.reshape(2, C, 1, CBLK),
      bts.reshape(2, C, 1, CBLK))


def _gat_body(z_ref, src_ref, dst_ref, emb_ref,
              wr_ref, wz_ref, wn_ref,
              bir_ref, biz_ref, bin_ref,
              bhr_ref, bhz_ref, bhn_ref,
              gatw_ref, atti_ref, attj_ref, gatb_ref,
              gnng_ref, gnnb_ref, bnog_ref, bnob_ref,
              outw_ref, outb_ref,
              out_ref,
              xp_s, emb_s, buf_s, st1_s, st2_s, A_s):
    ph = pl.program_id(0)
    b = pl.program_id(1)

    @pl.when((ph == 0) & (b == 0))
    def _init():
        for p in range(NPROC):
            e = emb_ref[p]
            for l in range(3):
                hs = []
                for dr in range(2):
                    idx = (p * 3 + l) * 2 + dr
                    gr = jax.lax.dot_general(
                        e, wr_ref[idx], (((1,), (1,)), ((), ())),
                        preferred_element_type=jnp.float32) + bir_ref[idx:idx + 1]
                    gz = jax.lax.dot_general(
                        e, wz_ref[idx], (((1,), (1,)), ((), ())),
                        preferred_element_type=jnp.float32) + biz_ref[idx:idx + 1]
                    gn = jax.lax.dot_general(
                        e, wn_ref[idx], (((1,), (1,)), ((), ())),
                        preferred_element_type=jnp.float32) + bin_ref[idx:idx + 1]
                    r = jax.nn.sigmoid(gr + bhr_ref[idx:idx + 1])
                    zg = jax.nn.sigmoid(gz + bhz_ref[idx:idx + 1])
                    nn_ = jnp.tanh(gn + r * bhn_ref[idx:idx + 1])
                    hs.append((1.0 - zg) * nn_)
                e = jnp.concatenate(hs, axis=1)
            emb_s[p * 64:(p + 1) * 64, :] = e
        st1_s[...] = jnp.zeros((2, DIM), jnp.float32)
        st2_s[...] = jnp.zeros((2, DIM), jnp.float32)

        def body(ch, acc):
            dsts = dst_ref[pl.ds(ch, 1), :]
            srcs = src_ref[pl.ds(ch, 1), :]
            rows = jax.lax.broadcasted_iota(jnp.int32, (N, ECHUNK), 0)
            ohd = (rows == dsts).astype(jnp.float32)
            ohs = (rows == srcs).astype(jnp.float32)
            return acc + jax.lax.dot_general(
                ohd, ohs, (((1,), (1,)), ((), ())),
                preferred_element_type=jnp.float32)

        A_s[...] = jax.lax.fori_loop(
            0, NCHUNK, body, jnp.zeros((N, N), jnp.float32))

    def _bclane(col, n):
        return jnp.broadcast_to(col, (col.shape[0], n))

    @pl.when(ph == 0)
    def _p0():
        zb = z_ref[pl.ds(b * N, N), :]
        xpb = jnp.dot(zb, gatw_ref[...], preferred_element_type=jnp.float32)
        xp_s[pl.ds(b * N, N), :] = xpb
        cat = jnp.concatenate([xpb, emb_s[...]], axis=1)
        ti = jax.lax.dot_general(cat, atti_ref[...], (((1,), (1,)), ((), ())),
                                 preferred_element_type=jnp.float32)
        tj = jax.lax.dot_general(attj_ref[...], cat, (((1,), (1,)), ((), ())),
                                 preferred_element_type=jnp.float32)
        t = _bclane(ti, N) + tj
        alpha = jnp.where(t >= 0, t, 0.2 * t)
        A = A_s[...]
        mask = A > 0
        am = jnp.max(jnp.where(mask, alpha, -1e30), axis=1, keepdims=True)
        am = jnp.where(am > -1e29, am, 0.0)
        P = A * jnp.where(mask, jnp.exp(alpha - _bclane(am, N)), 0.0)
        den = jnp.sum(P, axis=1, keepdims=True)
        agg = jnp.dot(P, xpb, preferred_element_type=jnp.float32,
                      precision=jax.lax.Precision.HIGHEST)
        aggu = agg / (_bclane(den, DIM) + 1e-16) + gatb_ref[...]
        buf_s[pl.ds(b * N, N), :] = aggu
        st1_s[0:1, :] += jnp.sum(aggu, axis=0, keepdims=True)

    cnt = float(B * N)

    @pl.when(ph == 1)
    def _p1v():
        m = st1_s[0:1, :] / cnt
        dev = buf_s[pl.ds(b * N, N), :] - m
        st1_s[1:2, :] += jnp.sum(dev * dev, axis=0, keepdims=True)

    @pl.when(ph == 2)
    def _p2():
        m = st1_s[0:1, :] / cnt
        v = st1_s[1:2, :] / cnt
        af = buf_s[pl.ds(b * N, N), :]
        gcn = jnp.maximum(
            (af - m) / jnp.sqrt(v + 1e-5) * gnng_ref[...] + gnnb_ref[...], 0.0)
        of = gcn * emb_s[...]
        buf_s[pl.ds(b * N, N), :] = of
        st2_s[0:1, :] += jnp.sum(of, axis=0, keepdims=True)

    @pl.when(ph == 3)
    def _p3v():
        m = st2_s[0:1, :] / cnt
        dev = buf_s[pl.ds(b * N, N), :] - m
        st2_s[1:2, :] += jnp.sum(dev * dev, axis=0, keepdims=True)

    @pl.when(ph == 4)
    def _p4():
        m = st2_s[0:1, :] / cnt
        v = st2_s[1:2, :] / cnt
        of = buf_s[pl.ds(b * N, N), :]
        o = jnp.maximum(
            (of - m) / jnp.sqrt(v + 1e-5) * bnog_ref[...] + bnob_ref[...], 0.0)
        res = jax.lax.dot_general(outw_ref[...], o, (((1,), (1,)), ((), ())),
                                  preferred_element_type=jnp.float32)
        out_ref[...] = (res + outb_ref[0, 0]).reshape(1, 1, N)


def _gat(z, src2, dst2, emb, wr, wz, wn, bir, biz, bin_, bhr, bhz, bhn,
         gatw, atti, attj, gatb, gnng, gnnb, bnog, bnob, outw, outb):
    full = lambda shape: pl.BlockSpec(shape, lambda ph, b: tuple(0 for _ in shape))
    G = NPROC * 3 * 2
    return pl.pallas_call(
        _gat_body,
        grid=(5, B),
        in_specs=[
            full((B * N, W)),
            full((NCHUNK, ECHUNK)),
            full((NCHUNK, ECHUNK)),
            full((NPROC, DIM, DIM)),
            full((G, H, DIM)), full((G, H, DIM)), full((G, H, DIM)),
            full((G, H)), full((G, H)), full((G, H)),
            full((G, H)), full((G, H)), full((G, H)),
            full((W, DIM)),
            full((1, 2 * DIM)), full((1, 2 * DIM)),
            full((1, DIM)),
            full((1, DIM)), full((1, DIM)), full((1, DIM)), full((1, DIM)),
            full((1, DIM)), full((1, 1)),
        ],
        out_specs=pl.BlockSpec((1, 1, N), lambda ph, b: (b, 0, 0)),
        out_shape=jax.ShapeDtypeStruct((B, 1, N), jnp.float32),
        scratch_shapes=[
            pltpu.VMEM((B * N, DIM), jnp.float32),
            pltpu.VMEM((N, DIM), jnp.float32),
            pltpu.VMEM((B * N, DIM), jnp.float32),
            pltpu.VMEM((2, DIM), jnp.float32),
            pltpu.VMEM((2, DIM), jnp.float32),
            pltpu.VMEM((N, N), jnp.float32),
        ],
    )(z, src2, dst2, emb, wr, wz, wn, bir, biz, bin_, bhr, bhz, bhn,
      gatw, atti, attj, gatb, gnng, gnnb, bnog, bnob, outw, outb)


def kernel(data, target, org_edge_index, emb_tables, gru_Wih, gru_Whh,
           gru_bih, gru_bhh, enc_W, enc_b, enc_g, enc_beta, dec_W, dec_b,
           dec_g, dec_beta, gat_W, att_i, att_j, gat_b, gnn_g, gnn_beta,
           bno_g, bno_beta, out_W, out_b):
    x = data.reshape(B, D)
    z = _ae(x, enc_W, enc_b, enc_g, enc_beta)
    xr = _ae(z, dec_W, dec_b, dec_g, dec_beta)

    eidx = org_edge_index.astype(jnp.int32)
    src2 = eidx[0].reshape(NCHUNK, ECHUNK)
    dst2 = eidx[1].reshape(NCHUNK, ECHUNK)

    G = NPROC * 3 * 2
    wih = gru_Wih.reshape(G, 3 * H, DIM)
    wr, wz, wn = wih[:, :H, :], wih[:, H:2 * H, :], wih[:, 2 * H:, :]
    bih = gru_bih.reshape(G, 3 * H)
    bir, biz, bin_ = bih[:, :H], bih[:, H:2 * H], bih[:, 2 * H:]
    bhh = gru_bhh.reshape(G, 3 * H)
    bhr, bhz, bhn = bhh[:, :H], bhh[:, H:2 * H], bhh[:, 2 * H:]

    atti = att_i.reshape(1, 2 * DIM)
    attj = att_j.reshape(1, 2 * DIM)

    out = _gat(z.reshape(B * N, W), src2, dst2, emb_tables,
               wr, wz, wn, bir, biz, bin_, bhr, bhz, bhn,
               gat_W, atti, attj, gat_b.reshape(1, DIM),
               gnn_g.reshape(1, DIM), gnn_beta.reshape(1, DIM),
               bno_g.reshape(1, DIM), bno_beta.reshape(1, DIM),
               out_W.reshape(1, DIM), out_b.reshape(1, 1))

    return (out.reshape(B, N), xr.reshape(B, N, W), z.reshape(B, N, W))
